# baseline (speedup 1.0000x reference)
import functools

import jax
import jax.numpy as jnp
from jax import lax
from jax.experimental import pallas as pl
from jax.experimental.pallas import tpu as pltpu
from jax.experimental.pallas import tpu_sc as plsc

B = 16
C1 = 768
C2 = 384
HW = 56 * 56

_NC = 2
_NS = 16
_NW = _NC * _NS
_ROWS = B * HW
_RPW = _ROWS // _NW
_R = 56
_NCHUNK = _RPW // _R
_L = 16
_NJ = C2 // _L


def _topk_body(y_ref, w_ref, idx_ref):
    yf = (y_ref[0].astype(jnp.bfloat16)).astype(jnp.float32)
    w0 = w_ref[0, 0].astype(jnp.bfloat16).astype(jnp.float32)
    w1 = w_ref[0, 1].astype(jnp.bfloat16).astype(jnp.float32)
    w2 = w_ref[0, 2].astype(jnp.bfloat16).astype(jnp.float32)
    z = jnp.zeros((1, 1), jnp.float32)
    y_lm1 = jnp.concatenate([z, yf[:, : C1 - 1]], axis=1)
    y_lp1 = jnp.concatenate([yf[:, 1:], z], axis=1)
    yc = (w0 * y_lm1 + w1 * yf) + w2 * y_lp1
    s_row = jnp.float32(1.0) / (jnp.float32(1.0) + jnp.exp(-yc))

    ii = lax.broadcasted_iota(jnp.int32, (C1, C1), 0)
    jj = lax.broadcasted_iota(jnp.int32, (C1, C1), 1)
    eye = (ii == jj).astype(jnp.float32)
    s_col = jnp.sum(eye * s_row, axis=1, keepdims=True)
    gt = s_row > s_col
    tie = (s_row == s_col) & (jj < ii)
    rank_col = jnp.sum((gt | tie).astype(jnp.int32), axis=1, keepdims=True)

    pp = lax.broadcasted_iota(jnp.int32, (C1, C2), 1)
    cc = lax.broadcasted_iota(jnp.int32, (C1, C2), 0)
    onehot = rank_col == pp
    idx_ref[0, 0, :] = jnp.sum(jnp.where(onehot, cc, 0), axis=0)


def _topk_idx(y3, w13):
    return pl.pallas_call(
        _topk_body,
        grid=(B,),
        in_specs=[
            pl.BlockSpec((1, 1, C1), lambda b: (b, 0, 0)),
            pl.BlockSpec(memory_space=pltpu.SMEM),
        ],
        out_specs=pl.BlockSpec((1, 1, C2), lambda b: (b, 0, 0)),
        out_shape=jax.ShapeDtypeStruct((B, 1, C2), jnp.int32),
    )(y3, w13)


def _gather_body(x_hbm, idx_hbm, out_hbm, idxv, in_a, in_b, outbuf,
                 sem_a, sem_b):
    cid = lax.axis_index("c")
    sid = lax.axis_index("s")
    wid = sid * _NC + cid
    row0 = wid * _RPW
    bat = wid // 2

    pltpu.sync_copy(idx_hbm.at[bat], idxv)
    chvs = [idxv[pl.ds(j * _L, _L)] for j in range(_NJ)]
    iota = lax.iota(jnp.int32, _L)

    def start_in(buf, sem, g):
        pltpu.async_copy(x_hbm.at[pl.ds(row0 + g * _R, _R)], buf, sem)

    start_in(in_a, sem_a, 0)
    start_in(in_b, sem_b, 1)

    def pair(h, carry):
        for buf, sem, off in ((in_a, sem_a, 0), (in_b, sem_b, 1)):
            g = 2 * h + off
            base = row0 + g * _R
            pltpu.make_async_copy(
                x_hbm.at[pl.ds(base, _R)], buf, sem).wait()

            @plsc.parallel_loop(0, _R, unroll=8)
            def row(r):
                rv = jnp.full((_L,), r, jnp.int32)
                for j in range(_NJ):
                    vals = plsc.load_gather(buf, [rv, chvs[j]])
                    plsc.store_scatter(outbuf, [rv, iota + (j * _L)], vals)
            pltpu.sync_copy(outbuf, out_hbm.at[pl.ds(base, _R)])

            @pl.when(g + 2 < _NCHUNK)
            def _():
                start_in(buf, sem, g + 2)
        return carry

    lax.fori_loop(0, _NCHUNK // 2, pair, 0)


@functools.lru_cache(maxsize=1)
def _gather_call():
    return pl.kernel(
        _gather_body,
        compiler_params=pltpu.CompilerParams(needs_layout_passes=False),
        out_type=jax.ShapeDtypeStruct((_ROWS, C2), jnp.float32),
        mesh=plsc.VectorSubcoreMesh(
            core_axis_name="c", subcore_axis_name="s",
            num_cores=_NC, num_subcores=_NS,
        ),
        scratch_types=[
            pltpu.VMEM((C2,), jnp.int32),
            pltpu.VMEM((_R, C1), jnp.float32),
            pltpu.VMEM((_R, C1), jnp.float32),
            pltpu.VMEM((_R, C2), jnp.float32),
            pltpu.SemaphoreType.DMA,
            pltpu.SemaphoreType.DMA,
        ],
    )


def kernel(x, W):
    y = jnp.mean(x, axis=(2, 3))
    idx3 = _topk_idx(y.reshape(B, 1, C1), W.reshape(1, 3))
    xt = jnp.transpose(x, (0, 2, 3, 1)).reshape(_ROWS, C1)
    out2 = _gather_call()(xt, idx3.reshape(B, C2))
    out = jnp.transpose(out2.reshape(B, HW, C2), (0, 2, 1))
    return out.reshape(B, C2, 56, 56)


# speedup vs baseline: 1.1597x; 1.1597x over previous
import functools

import jax
import jax.numpy as jnp
from jax import lax
from jax.experimental import pallas as pl
from jax.experimental.pallas import tpu as pltpu
from jax.experimental.pallas import tpu_sc as plsc

B = 16
C1 = 768
C2 = 384
HW = 56 * 56

_NC = 2
_NS = 16
_NW = _NC * _NS
_ROWS = B * HW
_RPW = _ROWS // _NW
_R = 56
_NCHUNK = _RPW // _R
_L = 16
_NJ = C2 // _L


def _topk_body(y_ref, w_ref, idx_ref):
    yf = (y_ref[0].astype(jnp.bfloat16)).astype(jnp.float32)
    w0 = w_ref[0, 0].astype(jnp.bfloat16).astype(jnp.float32)
    w1 = w_ref[0, 1].astype(jnp.bfloat16).astype(jnp.float32)
    w2 = w_ref[0, 2].astype(jnp.bfloat16).astype(jnp.float32)
    z = jnp.zeros((1, 1), jnp.float32)
    y_lm1 = jnp.concatenate([z, yf[:, : C1 - 1]], axis=1)
    y_lp1 = jnp.concatenate([yf[:, 1:], z], axis=1)
    yc = (w0 * y_lm1 + w1 * yf) + w2 * y_lp1
    s_row = jnp.float32(1.0) / (jnp.float32(1.0) + jnp.exp(-yc))

    ii = lax.broadcasted_iota(jnp.int32, (C1, C1), 0)
    jj = lax.broadcasted_iota(jnp.int32, (C1, C1), 1)
    eye = (ii == jj).astype(jnp.float32)
    s_col = jnp.sum(eye * s_row, axis=1, keepdims=True)
    gt = s_row > s_col
    tie = (s_row == s_col) & (jj < ii)
    rank_col = jnp.sum((gt | tie).astype(jnp.int32), axis=1, keepdims=True)

    pp = lax.broadcasted_iota(jnp.int32, (C1, C2), 1)
    cc = lax.broadcasted_iota(jnp.int32, (C1, C2), 0)
    onehot = rank_col == pp
    idx_ref[0, 0, :] = jnp.sum(jnp.where(onehot, cc, 0), axis=0)


def _topk_idx(y3, w13):
    return pl.pallas_call(
        _topk_body,
        grid=(B,),
        in_specs=[
            pl.BlockSpec((1, 1, C1), lambda b: (b, 0, 0)),
            pl.BlockSpec(memory_space=pltpu.SMEM),
        ],
        out_specs=pl.BlockSpec((1, 1, C2), lambda b: (b, 0, 0)),
        out_shape=jax.ShapeDtypeStruct((B, 1, C2), jnp.int32),
    )(y3, w13)


def _gather_body(x_hbm, idx_hbm, out_hbm, idxv, in_a, in_b, out_a, out_b,
                 sem_a, sem_b, sem_oa, sem_ob):
    cid = lax.axis_index("c")
    sid = lax.axis_index("s")
    wid = sid * _NC + cid
    row0 = wid * _RPW
    bat = wid // 2

    pltpu.sync_copy(idx_hbm.at[bat], idxv)
    chvs = [idxv[pl.ds(j * _L, _L)] for j in range(_NJ)]
    iota = lax.iota(jnp.int32, _L)

    def start_in(buf, sem, g):
        pltpu.async_copy(x_hbm.at[pl.ds(row0 + g * _R, _R)], buf, sem)

    start_in(in_a, sem_a, 0)
    start_in(in_b, sem_b, 1)

    def pair(h, carry):
        for buf, sem, outb, semo, off in (
                (in_a, sem_a, out_a, sem_oa, 0),
                (in_b, sem_b, out_b, sem_ob, 1)):
            g = 2 * h + off
            base = row0 + g * _R
            pltpu.make_async_copy(
                x_hbm.at[pl.ds(base, _R)], buf, sem).wait()

            @pl.when(g >= 2)
            def _():
                pltpu.make_async_copy(
                    outb, out_hbm.at[pl.ds(base, _R)], semo).wait()

            @plsc.parallel_loop(0, _R, unroll=4)
            def row(r):
                rv = jnp.full((_L,), r, jnp.int32)
                for j in range(_NJ):
                    vals = plsc.load_gather(buf, [rv, chvs[j]])
                    plsc.store_scatter(outb, [rv, iota + (j * _L)], vals)
            pltpu.async_copy(outb, out_hbm.at[pl.ds(base, _R)], semo)

            @pl.when(g + 2 < _NCHUNK)
            def _():
                start_in(buf, sem, g + 2)
        return carry

    lax.fori_loop(0, _NCHUNK // 2, pair, 0)
    last = row0 + (_NCHUNK - 1) * _R
    pltpu.make_async_copy(out_a, out_hbm.at[pl.ds(last, _R)], sem_oa).wait()
    pltpu.make_async_copy(out_b, out_hbm.at[pl.ds(last, _R)], sem_ob).wait()


@functools.lru_cache(maxsize=1)
def _gather_call():
    return pl.kernel(
        _gather_body,
        compiler_params=pltpu.CompilerParams(needs_layout_passes=False),
        out_type=jax.ShapeDtypeStruct((_ROWS, C2), jnp.float32),
        mesh=plsc.VectorSubcoreMesh(
            core_axis_name="c", subcore_axis_name="s",
            num_cores=_NC, num_subcores=_NS,
        ),
        scratch_types=[
            pltpu.VMEM((C2,), jnp.int32),
            pltpu.VMEM((_R, C1), jnp.float32),
            pltpu.VMEM((_R, C1), jnp.float32),
            pltpu.VMEM((_R, C2), jnp.float32),
            pltpu.VMEM((_R, C2), jnp.float32),
            pltpu.SemaphoreType.DMA,
            pltpu.SemaphoreType.DMA,
            pltpu.SemaphoreType.DMA,
            pltpu.SemaphoreType.DMA,
        ],
    )


def kernel(x, W):
    y = jnp.mean(x, axis=(2, 3))
    idx3 = _topk_idx(y.reshape(B, 1, C1), W.reshape(1, 3))
    xt = jnp.transpose(x, (0, 2, 3, 1)).reshape(_ROWS, C1)
    out2 = _gather_call()(xt, idx3.reshape(B, C2))
    out = jnp.transpose(out2.reshape(B, HW, C2), (0, 2, 1))
    return out.reshape(B, C2, 56, 56)
